# Initial kernel scaffold; baseline (speedup 1.0000x reference)
import functools

import jax
import jax.numpy as jnp
from jax import lax
from jax.experimental import pallas as pl
from jax.experimental.pallas import tpu as pltpu
from jax.experimental.pallas import tpu_sc as plsc

DB = 32
NC = 2
NS = 16
LANES = 16
BATCH = 128
CHUNK = 2048
CB = CHUNK // BATCH


def _proj_body(x_ref, w_ref, o_ref):
    o_ref[...] = jnp.dot(x_ref[...], w_ref[...],
                         preferred_element_type=jnp.float32)


def _post_body(xo_ref, sp_ref, sn_ref, wm_ref, g_ref, b_ref, a_ref, o_ref):
    n = xo_ref.shape[0]
    a = a_ref[0]

    def bn_prelu(v, j):
        g = g_ref[j, :]
        b = b_ref[j, :]
        mean = jnp.sum(v, axis=0, keepdims=True) * (1.0 / n)
        var = jnp.sum(v * v, axis=0, keepdims=True) * (1.0 / n) - mean * mean
        y = g * (v - mean) * jax.lax.rsqrt(var + 1e-5) + b
        return jnp.where(y >= 0, y, a * y)

    cat = jnp.concatenate(
        [bn_prelu(xo_ref[...], 0),
         bn_prelu(sp_ref[...], 1),
         bn_prelu(sn_ref[...], 2)], axis=1)
    e = jnp.dot(cat, wm_ref[...], preferred_element_type=jnp.float32)
    nrm = jnp.sqrt(jnp.sum(e * e, axis=1, keepdims=True))
    o_ref[...] = e / jnp.maximum(nrm, 1e-12)


def _spmm_sc_body(n, ep, xpn_hbm, idx_hbm, w_hbm, z_hbm, out_hbm,
                  col_v, row_v, w_v, gath_v, acc_s, gsem, ssem):
    c = lax.axis_index("c")
    s = lax.axis_index("s")

    zr = n // NS
    pltpu.sync_copy(z_hbm.at[pl.ds(s * zr, zr)], acc_s.at[pl.ds(s * zr, zr)])
    plsc.subcore_barrier()

    tile_edges = ep // NS
    n_chunks = tile_edges // CHUNK
    base_b = s * (tile_edges // BATCH)

    def chunk_body(k, carry):
        boff = base_b + k * CB
        eoff = s * tile_edges + k * CHUNK
        pltpu.sync_copy(idx_hbm.at[c, 1, pl.ds(boff, CB)], col_v)
        pltpu.sync_copy(idx_hbm.at[c, 0, pl.ds(boff, CB)], row_v)
        pltpu.sync_copy(w_hbm.at[c, pl.ds(eoff, CHUNK)], w_v)

        src = xpn_hbm.at[c]
        gd = [pltpu.async_copy(src.at[col_v.at[j]],
                               gath_v.at[pl.ds(j * BATCH, BATCH)], gsem)
              for j in range(CB)]
        for d in gd:
            d.wait()

        def scale_body(g, carry2):
            wgrp = w_v[pl.ds(g * LANES, LANES)]
            for e in range(LANES):
                ws = jnp.take(wgrp, jnp.full((LANES,), e, jnp.int32),
                              mode="promise_in_bounds")
                r = g * LANES + e
                gath_v[r, 0:16] = gath_v[r, 0:16] * ws
                gath_v[r, 16:32] = gath_v[r, 16:32] * ws
            return carry2

        lax.fori_loop(0, CHUNK // LANES, scale_body, 0)

        sd = [pltpu.async_copy(gath_v.at[pl.ds(j * BATCH, BATCH)],
                               acc_s.at[row_v.at[j]], ssem, add=True)
              for j in range(CB)]
        for d in sd:
            d.wait()
        return carry

    lax.fori_loop(0, n_chunks, chunk_body, 0)
    plsc.subcore_barrier()

    pltpu.sync_copy(acc_s.at[pl.ds(s * zr, zr)],
                    out_hbm.at[c, pl.ds(s * zr, zr)])


def kernel(x, pos_index, pos_weight, neg_index, neg_weight, other_index,
           other_weight, W_org, W_pos, W_neg, W_mlp, g_org, b_org, g_pos,
           b_pos, g_neg, b_neg, prelu_a):
    n, da = x.shape
    e = pos_index.shape[1]
    f32 = jnp.float32

    wcat = jnp.concatenate([W_org, W_pos, W_neg], axis=1)
    cat = pl.pallas_call(
        _proj_body,
        out_shape=jax.ShapeDtypeStruct((n, 3 * DB), f32),
    )(x, wcat)
    xo = cat[:, 0:DB]
    xpn = jnp.stack([cat[:, DB:2 * DB], cat[:, 2 * DB:3 * DB]])

    step = NS * CHUNK
    ep = ((e + step - 1) // step) * step
    pad = ep - e
    idx = jnp.stack([
        jnp.pad(pos_index, ((0, 0), (0, pad))),
        jnp.pad(neg_index, ((0, 0), (0, pad))),
    ]).reshape(2, 2, ep // BATCH, BATCH)
    wst = jnp.stack([
        jnp.pad(pos_weight, (0, pad)),
        jnp.pad(neg_weight, (0, pad)),
    ])
    zeros = jnp.zeros((n, DB), f32)

    mesh = plsc.VectorSubcoreMesh(core_axis_name="c", subcore_axis_name="s")
    spmm = pl.kernel(
        functools.partial(_spmm_sc_body, n, ep),
        out_type=jax.ShapeDtypeStruct((2, n, DB), f32),
        mesh=mesh,
        scratch_types=[
            pltpu.VMEM((CB, BATCH), jnp.int32),
            pltpu.VMEM((CB, BATCH), jnp.int32),
            pltpu.VMEM((CHUNK,), f32),
            pltpu.VMEM((CHUNK, DB), f32),
            pltpu.VMEM_SHARED((n, DB), f32),
            pltpu.SemaphoreType.DMA,
            pltpu.SemaphoreType.DMA,
        ],
    )
    seg = spmm(xpn, idx, wst, zeros)

    gs = jnp.stack([g_org, g_pos, g_neg]).reshape(3, DB)
    bs = jnp.stack([b_org, b_pos, b_neg]).reshape(3, DB)
    embs = pl.pallas_call(
        _post_body,
        out_shape=jax.ShapeDtypeStruct((n, DB), f32),
        in_specs=[
            pl.BlockSpec(memory_space=pltpu.VMEM),
            pl.BlockSpec(memory_space=pltpu.VMEM),
            pl.BlockSpec(memory_space=pltpu.VMEM),
            pl.BlockSpec(memory_space=pltpu.VMEM),
            pl.BlockSpec(memory_space=pltpu.VMEM),
            pl.BlockSpec(memory_space=pltpu.VMEM),
            pl.BlockSpec(memory_space=pltpu.SMEM),
        ],
    )(xo, seg[0], seg[1], W_mlp, gs, bs, prelu_a.reshape(1))
    return embs


# speedup vs baseline: 9.7243x; 9.7243x over previous
import functools

import jax
import jax.numpy as jnp
from jax import lax
from jax.experimental import pallas as pl
from jax.experimental.pallas import tpu as pltpu
from jax.experimental.pallas import tpu_sc as plsc

DB = 32
NC = 2
NS = 16
LANES = 16
BATCH = 128
CHUNK = 2048
CB = CHUNK // BATCH


def _vbroadcast(vec, lane):
    idx = jnp.full((LANES, 1), lane, jnp.int32)
    return lax.gather(
        vec, idx,
        lax.GatherDimensionNumbers(offset_dims=(), collapsed_slice_dims=(0,),
                                   start_index_map=(0,)),
        (1,), mode=lax.GatherScatterMode.PROMISE_IN_BOUNDS)


def _proj_body(x_ref, w_ref, o_ref):
    o_ref[...] = jnp.dot(x_ref[...], w_ref[...],
                         preferred_element_type=jnp.float32)


def _post_body(xo_ref, sp_ref, sn_ref, wm_ref, g_ref, b_ref, a_ref, o_ref):
    n = xo_ref.shape[0]
    a = a_ref[0]

    def bn_prelu(v, j):
        g = g_ref[j, :]
        b = b_ref[j, :]
        mean = jnp.sum(v, axis=0, keepdims=True) * (1.0 / n)
        var = jnp.sum(v * v, axis=0, keepdims=True) * (1.0 / n) - mean * mean
        y = g * (v - mean) * jax.lax.rsqrt(var + 1e-5) + b
        return jnp.where(y >= 0, y, a * y)

    cat = jnp.concatenate(
        [bn_prelu(xo_ref[...], 0),
         bn_prelu(sp_ref[...], 1),
         bn_prelu(sn_ref[...], 2)], axis=1)
    e = jnp.dot(cat, wm_ref[...], preferred_element_type=jnp.float32)
    nrm = jnp.sqrt(jnp.sum(e * e, axis=1, keepdims=True))
    o_ref[...] = e / jnp.maximum(nrm, 1e-12)


def _spmm_sc_body(n, ep, xpn_hbm, idx_hbm, w_hbm, z_hbm, out_hbm,
                  col_v, row_v, w_v, gath_v, acc_s, gsem, ssem):
    c = lax.axis_index("c")
    s = lax.axis_index("s")

    zr = n // NS
    pltpu.sync_copy(z_hbm.at[pl.ds(s * zr, zr)], acc_s.at[pl.ds(s * zr, zr)])
    plsc.subcore_barrier()

    tile_edges = ep // NS
    n_chunks = tile_edges // CHUNK
    base_b = s * (tile_edges // BATCH)

    def chunk_body(k, carry):
        boff = base_b + k * CB
        eoff = s * tile_edges + k * CHUNK
        pltpu.sync_copy(idx_hbm.at[c, 1, pl.ds(boff, CB)], col_v)
        pltpu.sync_copy(idx_hbm.at[c, 0, pl.ds(boff, CB)], row_v)
        pltpu.sync_copy(w_hbm.at[c, pl.ds(eoff, CHUNK)], w_v)

        src = xpn_hbm.at[c]
        gd = [pltpu.async_copy(src.at[col_v.at[j]],
                               gath_v.at[pl.ds(j * BATCH, BATCH)], gsem)
              for j in range(CB)]
        for d in gd:
            d.wait()

        def scale_body(g, carry2):
            wgrp = w_v[pl.ds(g * LANES, LANES)]
            for e in range(LANES):
                ws = _vbroadcast(wgrp, e)
                r = g * LANES + e
                gath_v[r, 0:16] = gath_v[r, 0:16] * ws
                gath_v[r, 16:32] = gath_v[r, 16:32] * ws
            return carry2

        lax.fori_loop(0, CHUNK // LANES, scale_body, 0)

        sd = [pltpu.async_copy(gath_v.at[pl.ds(j * BATCH, BATCH)],
                               acc_s.at[row_v.at[j]], ssem, add=True)
              for j in range(CB)]
        for d in sd:
            d.wait()
        return carry

    lax.fori_loop(0, n_chunks, chunk_body, 0)
    plsc.subcore_barrier()

    pltpu.sync_copy(acc_s.at[pl.ds(s * zr, zr)],
                    out_hbm.at[c, pl.ds(s * zr, zr)])


def kernel(x, pos_index, pos_weight, neg_index, neg_weight, other_index,
           other_weight, W_org, W_pos, W_neg, W_mlp, g_org, b_org, g_pos,
           b_pos, g_neg, b_neg, prelu_a):
    n, da = x.shape
    e = pos_index.shape[1]
    f32 = jnp.float32

    wcat = jnp.concatenate([W_org, W_pos, W_neg], axis=1)
    cat = pl.pallas_call(
        _proj_body,
        out_shape=jax.ShapeDtypeStruct((n, 3 * DB), f32),
    )(x, wcat)
    xo = cat[:, 0:DB]
    xpn = jnp.stack([cat[:, DB:2 * DB], cat[:, 2 * DB:3 * DB]])

    step = NS * CHUNK
    ep = ((e + step - 1) // step) * step
    pad = ep - e
    idx = jnp.stack([
        jnp.pad(pos_index, ((0, 0), (0, pad))),
        jnp.pad(neg_index, ((0, 0), (0, pad))),
    ]).reshape(2, 2, ep // BATCH, BATCH)
    wst = jnp.stack([
        jnp.pad(pos_weight, (0, pad)),
        jnp.pad(neg_weight, (0, pad)),
    ])
    zeros = jnp.zeros((n, DB), f32)

    mesh = plsc.VectorSubcoreMesh(core_axis_name="c", subcore_axis_name="s")
    spmm = pl.kernel(
        functools.partial(_spmm_sc_body, n, ep),
        out_type=jax.ShapeDtypeStruct((2, n, DB), f32),
        mesh=mesh,
        scratch_types=[
            pltpu.VMEM((CB, BATCH), jnp.int32),
            pltpu.VMEM((CB, BATCH), jnp.int32),
            pltpu.VMEM((CHUNK,), f32),
            pltpu.VMEM((CHUNK, DB), f32),
            pltpu.VMEM_SHARED((n, DB), f32),
            pltpu.SemaphoreType.DMA,
            pltpu.SemaphoreType.DMA,
        ],
        compiler_params=pltpu.CompilerParams(use_tc_tiling_on_sc=False),
    )
    seg = spmm(xpn, idx, wst, zeros)

    gs = jnp.stack([g_org, g_pos, g_neg]).reshape(3, DB)
    bs = jnp.stack([b_org, b_pos, b_neg]).reshape(3, DB)
    embs = pl.pallas_call(
        _post_body,
        out_shape=jax.ShapeDtypeStruct((n, DB), f32),
        in_specs=[
            pl.BlockSpec(memory_space=pltpu.VMEM),
            pl.BlockSpec(memory_space=pltpu.VMEM),
            pl.BlockSpec(memory_space=pltpu.VMEM),
            pl.BlockSpec(memory_space=pltpu.VMEM),
            pl.BlockSpec(memory_space=pltpu.VMEM),
            pl.BlockSpec(memory_space=pltpu.VMEM),
            pl.BlockSpec(memory_space=pltpu.SMEM),
        ],
    )(xo, seg[0], seg[1], W_mlp, gs, bs, prelu_a.reshape(1))
    return embs


# speedup vs baseline: 10.6832x; 1.0986x over previous
import functools

import jax
import jax.numpy as jnp
from jax import lax
from jax.experimental import pallas as pl
from jax.experimental.pallas import tpu as pltpu
from jax.experimental.pallas import tpu_sc as plsc

DB = 32
NC = 2
NS = 16
LANES = 16
BATCH = 128
CHUNK = 2048
CB = CHUNK // BATCH


def _vbroadcast(vec, lane):
    idx = jnp.full((LANES, 1), lane, jnp.int32)
    return lax.gather(
        vec, idx,
        lax.GatherDimensionNumbers(offset_dims=(), collapsed_slice_dims=(0,),
                                   start_index_map=(0,)),
        (1,), mode=lax.GatherScatterMode.PROMISE_IN_BOUNDS)


def _proj_body(x_ref, w_ref, o_ref):
    o_ref[...] = jnp.dot(x_ref[...], w_ref[...],
                         preferred_element_type=jnp.float32)


def _post_body(xo_ref, sp_ref, sn_ref, wm_ref, g_ref, b_ref, a_ref, o_ref):
    n = xo_ref.shape[0]
    a = a_ref[0]

    def bn_prelu(v, j):
        g = g_ref[j, :]
        b = b_ref[j, :]
        mean = jnp.sum(v, axis=0, keepdims=True) * (1.0 / n)
        var = jnp.sum(v * v, axis=0, keepdims=True) * (1.0 / n) - mean * mean
        y = g * (v - mean) * jax.lax.rsqrt(var + 1e-5) + b
        return jnp.where(y >= 0, y, a * y)

    cat = jnp.concatenate(
        [bn_prelu(xo_ref[...], 0),
         bn_prelu(sp_ref[...], 1),
         bn_prelu(sn_ref[...], 2)], axis=1)
    e = jnp.dot(cat, wm_ref[...], preferred_element_type=jnp.float32)
    nrm = jnp.sqrt(jnp.sum(e * e, axis=1, keepdims=True))
    o_ref[...] = e / jnp.maximum(nrm, 1e-12)


def _spmm_sc_body(n, ep, xpn_hbm, idx_hbm, w_hbm, z_hbm, out_hbm,
                  col_v, row_v, w_v, gath_v, acc_s, gsem, ssem):
    c = lax.axis_index("c")
    s = lax.axis_index("s")

    zr = n // NS
    pltpu.sync_copy(z_hbm.at[pl.ds(s * zr, zr)], acc_s.at[pl.ds(s * zr, zr)])
    plsc.subcore_barrier()

    tile_edges = ep // NS
    n_chunks = tile_edges // CHUNK
    base_b = s * (tile_edges // BATCH)

    def chunk_body(k, carry):
        eoff = s * tile_edges + k * CHUNK
        pltpu.sync_copy(idx_hbm.at[c, 1, pl.ds(eoff, CHUNK)], col_v)
        pltpu.sync_copy(idx_hbm.at[c, 0, pl.ds(eoff, CHUNK)], row_v)
        pltpu.sync_copy(w_hbm.at[c, pl.ds(eoff, CHUNK)], w_v)

        src = xpn_hbm.at[c]
        pltpu.async_copy(src.at[col_v], gath_v, gsem).wait()

        def scale_body(g, carry2):
            wgrp = w_v[pl.ds(g * LANES, LANES)]
            for e in range(LANES):
                ws = _vbroadcast(wgrp, e)
                r = g * LANES + e
                gath_v[r, 0:16] = gath_v[r, 0:16] * ws
                gath_v[r, 16:32] = gath_v[r, 16:32] * ws
            return carry2

        lax.fori_loop(0, CHUNK // LANES, scale_body, 0)

        pltpu.async_copy(gath_v, acc_s.at[row_v], ssem, add=True).wait()
        return carry

    lax.fori_loop(0, n_chunks, chunk_body, 0)
    plsc.subcore_barrier()

    pltpu.sync_copy(acc_s.at[pl.ds(s * zr, zr)],
                    out_hbm.at[c, pl.ds(s * zr, zr)])


def kernel(x, pos_index, pos_weight, neg_index, neg_weight, other_index,
           other_weight, W_org, W_pos, W_neg, W_mlp, g_org, b_org, g_pos,
           b_pos, g_neg, b_neg, prelu_a):
    n, da = x.shape
    e = pos_index.shape[1]
    f32 = jnp.float32

    wcat = jnp.concatenate([W_org, W_pos, W_neg], axis=1)
    cat = pl.pallas_call(
        _proj_body,
        out_shape=jax.ShapeDtypeStruct((n, 3 * DB), f32),
    )(x, wcat)
    xo = cat[:, 0:DB]
    xpn = jnp.stack([cat[:, DB:2 * DB], cat[:, 2 * DB:3 * DB]])

    step = NS * CHUNK
    ep = ((e + step - 1) // step) * step
    pad = ep - e
    idx = jnp.stack([
        jnp.pad(pos_index, ((0, 0), (0, pad))),
        jnp.pad(neg_index, ((0, 0), (0, pad))),
    ])
    wst = jnp.stack([
        jnp.pad(pos_weight, (0, pad)),
        jnp.pad(neg_weight, (0, pad)),
    ])
    zeros = jnp.zeros((n, DB), f32)

    mesh = plsc.VectorSubcoreMesh(core_axis_name="c", subcore_axis_name="s")
    spmm = pl.kernel(
        functools.partial(_spmm_sc_body, n, ep),
        out_type=jax.ShapeDtypeStruct((2, n, DB), f32),
        mesh=mesh,
        scratch_types=[
            pltpu.VMEM((CHUNK,), jnp.int32),
            pltpu.VMEM((CHUNK,), jnp.int32),
            pltpu.VMEM((CHUNK,), f32),
            pltpu.VMEM((CHUNK, DB), f32),
            pltpu.VMEM_SHARED((n, DB), f32),
            pltpu.SemaphoreType.DMA,
            pltpu.SemaphoreType.DMA,
        ],
        compiler_params=pltpu.CompilerParams(use_tc_tiling_on_sc=False),
    )
    seg = spmm(xpn, idx, wst, zeros)

    gs = jnp.stack([g_org, g_pos, g_neg]).reshape(3, DB)
    bs = jnp.stack([b_org, b_pos, b_neg]).reshape(3, DB)
    embs = pl.pallas_call(
        _post_body,
        out_shape=jax.ShapeDtypeStruct((n, DB), f32),
        in_specs=[
            pl.BlockSpec(memory_space=pltpu.VMEM),
            pl.BlockSpec(memory_space=pltpu.VMEM),
            pl.BlockSpec(memory_space=pltpu.VMEM),
            pl.BlockSpec(memory_space=pltpu.VMEM),
            pl.BlockSpec(memory_space=pltpu.VMEM),
            pl.BlockSpec(memory_space=pltpu.VMEM),
            pl.BlockSpec(memory_space=pltpu.SMEM),
        ],
    )(xo, seg[0], seg[1], W_mlp, gs, bs, prelu_a.reshape(1))
    return embs


# speedup vs baseline: 15.6862x; 1.4683x over previous
import functools

import jax
import jax.numpy as jnp
from jax import lax
from jax.experimental import pallas as pl
from jax.experimental.pallas import tpu as pltpu
from jax.experimental.pallas import tpu_sc as plsc

DB = 32
NC = 2
NS = 16
LANES = 16
BATCH = 128
CHUNK = 2048
CB = CHUNK // BATCH


def _vbroadcast(vec, lane):
    idx = jnp.full((LANES, 1), lane, jnp.int32)
    return lax.gather(
        vec, idx,
        lax.GatherDimensionNumbers(offset_dims=(), collapsed_slice_dims=(0,),
                                   start_index_map=(0,)),
        (1,), mode=lax.GatherScatterMode.PROMISE_IN_BOUNDS)


def _proj_body(x_ref, w_ref, o_ref):
    o_ref[...] = jnp.dot(x_ref[...], w_ref[...],
                         preferred_element_type=jnp.float32)


def _post_body(xo_ref, sp_ref, sn_ref, wm_ref, g_ref, b_ref, a_ref, o_ref):
    n = xo_ref.shape[0]
    a = a_ref[0]

    def bn_prelu(v, j):
        g = g_ref[j, :]
        b = b_ref[j, :]
        mean = jnp.sum(v, axis=0, keepdims=True) * (1.0 / n)
        var = jnp.sum(v * v, axis=0, keepdims=True) * (1.0 / n) - mean * mean
        y = g * (v - mean) * jax.lax.rsqrt(var + 1e-5) + b
        return jnp.where(y >= 0, y, a * y)

    cat = jnp.concatenate(
        [bn_prelu(xo_ref[...], 0),
         bn_prelu(sp_ref[...], 1),
         bn_prelu(sn_ref[...], 2)], axis=1)
    e = jnp.dot(cat, wm_ref[...], preferred_element_type=jnp.float32)
    nrm = jnp.sqrt(jnp.sum(e * e, axis=1, keepdims=True))
    o_ref[...] = e / jnp.maximum(nrm, 1e-12)


def _spmm_sc_body(n, ep, xpn_hbm, idx_hbm, w_hbm, z_hbm, out_hbm,
                  col_v, row_v, w_v, gath_v, acc_s, xs_s, gsem, ssem):
    c = lax.axis_index("c")
    s = lax.axis_index("s")

    zr = n // NS
    pltpu.sync_copy(z_hbm.at[pl.ds(s * zr, zr)], acc_s.at[pl.ds(s * zr, zr)])
    pltpu.sync_copy(xpn_hbm.at[c, pl.ds(s * zr, zr)],
                    xs_s.at[pl.ds(s * zr, zr)])
    plsc.subcore_barrier()

    tile_edges = ep // NS
    n_chunks = tile_edges // CHUNK
    base_b = s * (tile_edges // BATCH)

    def chunk_body(k, carry):
        eoff = s * tile_edges + k * CHUNK
        pltpu.sync_copy(idx_hbm.at[c, 1, pl.ds(eoff, CHUNK)], col_v)
        pltpu.sync_copy(idx_hbm.at[c, 0, pl.ds(eoff, CHUNK)], row_v)
        pltpu.sync_copy(w_hbm.at[c, pl.ds(eoff, CHUNK)], w_v)

        pltpu.async_copy(xs_s.at[col_v], gath_v, gsem).wait()

        def scale_body(g, carry2):
            wgrp = w_v[pl.ds(g * LANES, LANES)]
            for e in range(LANES):
                ws = _vbroadcast(wgrp, e)
                r = g * LANES + e
                gath_v[r, 0:16] = gath_v[r, 0:16] * ws
                gath_v[r, 16:32] = gath_v[r, 16:32] * ws
            return carry2

        lax.fori_loop(0, CHUNK // LANES, scale_body, 0)

        pltpu.async_copy(gath_v, acc_s.at[row_v], ssem, add=True).wait()
        return carry

    lax.fori_loop(0, n_chunks, chunk_body, 0)
    plsc.subcore_barrier()

    pltpu.sync_copy(acc_s.at[pl.ds(s * zr, zr)],
                    out_hbm.at[c, pl.ds(s * zr, zr)])


def kernel(x, pos_index, pos_weight, neg_index, neg_weight, other_index,
           other_weight, W_org, W_pos, W_neg, W_mlp, g_org, b_org, g_pos,
           b_pos, g_neg, b_neg, prelu_a):
    n, da = x.shape
    e = pos_index.shape[1]
    f32 = jnp.float32

    wcat = jnp.concatenate([W_org, W_pos, W_neg], axis=1)
    cat = pl.pallas_call(
        _proj_body,
        out_shape=jax.ShapeDtypeStruct((n, 3 * DB), f32),
    )(x, wcat)
    xo = cat[:, 0:DB]
    xpn = jnp.stack([cat[:, DB:2 * DB], cat[:, 2 * DB:3 * DB]])

    step = NS * CHUNK
    ep = ((e + step - 1) // step) * step
    pad = ep - e
    idx = jnp.stack([
        jnp.pad(pos_index, ((0, 0), (0, pad))),
        jnp.pad(neg_index, ((0, 0), (0, pad))),
    ])
    wst = jnp.stack([
        jnp.pad(pos_weight, (0, pad)),
        jnp.pad(neg_weight, (0, pad)),
    ])
    zeros = jnp.zeros((n, DB), f32)

    mesh = plsc.VectorSubcoreMesh(core_axis_name="c", subcore_axis_name="s")
    spmm = pl.kernel(
        functools.partial(_spmm_sc_body, n, ep),
        out_type=jax.ShapeDtypeStruct((2, n, DB), f32),
        mesh=mesh,
        scratch_types=[
            pltpu.VMEM((CHUNK,), jnp.int32),
            pltpu.VMEM((CHUNK,), jnp.int32),
            pltpu.VMEM((CHUNK,), f32),
            pltpu.VMEM((CHUNK, DB), f32),
            pltpu.VMEM_SHARED((n, DB), f32),
            pltpu.VMEM_SHARED((n, DB), f32),
            pltpu.SemaphoreType.DMA,
            pltpu.SemaphoreType.DMA,
        ],
        compiler_params=pltpu.CompilerParams(use_tc_tiling_on_sc=False),
    )
    seg = spmm(xpn, idx, wst, zeros)

    gs = jnp.stack([g_org, g_pos, g_neg]).reshape(3, DB)
    bs = jnp.stack([b_org, b_pos, b_neg]).reshape(3, DB)
    embs = pl.pallas_call(
        _post_body,
        out_shape=jax.ShapeDtypeStruct((n, DB), f32),
        in_specs=[
            pl.BlockSpec(memory_space=pltpu.VMEM),
            pl.BlockSpec(memory_space=pltpu.VMEM),
            pl.BlockSpec(memory_space=pltpu.VMEM),
            pl.BlockSpec(memory_space=pltpu.VMEM),
            pl.BlockSpec(memory_space=pltpu.VMEM),
            pl.BlockSpec(memory_space=pltpu.VMEM),
            pl.BlockSpec(memory_space=pltpu.SMEM),
        ],
    )(xo, seg[0], seg[1], W_mlp, gs, bs, prelu_a.reshape(1))
    return embs


# speedup vs baseline: 16.4045x; 1.0458x over previous
import functools

import jax
import jax.numpy as jnp
from jax import lax
from jax.experimental import pallas as pl
from jax.experimental.pallas import tpu as pltpu
from jax.experimental.pallas import tpu_sc as plsc

DB = 32
NS = 16
LANES = 16
CHUNK = 2048


def _vbroadcast(vec, lane):
    idx = jnp.full((LANES, 1), lane, jnp.int32)
    return lax.gather(
        vec, idx,
        lax.GatherDimensionNumbers(offset_dims=(), collapsed_slice_dims=(0,),
                                   start_index_map=(0,)),
        (1,), mode=lax.GatherScatterMode.PROMISE_IN_BOUNDS)


def _proj_body(x_ref, w_ref, xo_ref, xpn_ref):
    cat = jnp.dot(x_ref[...], w_ref[...], preferred_element_type=jnp.float32)
    xo_ref[...] = cat[:, 0:DB]
    xpn_ref[0] = cat[:, DB:2 * DB]
    xpn_ref[1] = cat[:, 2 * DB:3 * DB]


def _post_body(xo_ref, sp_ref, sn_ref, wm_ref, g_ref, b_ref, a_ref, o_ref):
    n = xo_ref.shape[0]
    a = a_ref[0]

    def bn_prelu(v, j):
        g = g_ref[j, :]
        b = b_ref[j, :]
        mean = jnp.sum(v, axis=0, keepdims=True) * (1.0 / n)
        var = jnp.sum(v * v, axis=0, keepdims=True) * (1.0 / n) - mean * mean
        y = g * (v - mean) * jax.lax.rsqrt(var + 1e-5) + b
        return jnp.where(y >= 0, y, a * y)

    cat = jnp.concatenate(
        [bn_prelu(xo_ref[...], 0),
         bn_prelu(sp_ref[...], 1),
         bn_prelu(sn_ref[...], 2)], axis=1)
    e = jnp.dot(cat, wm_ref[...], preferred_element_type=jnp.float32)
    nrm = jnp.sqrt(jnp.sum(e * e, axis=1, keepdims=True))
    o_ref[...] = e / jnp.maximum(nrm, 1e-12)


def _spmm_sc_body(n, ep, xpn_hbm, pidx_hbm, pw_hbm, nidx_hbm, nw_hbm, z_hbm,
                  out_hbm, col_v, row_v, w_v, gath_v, acc_s, xs_s, gsem, ssem):
    c = lax.axis_index("c")
    s = lax.axis_index("s")

    zr = n // NS
    pltpu.sync_copy(z_hbm.at[pl.ds(s * zr, zr)], acc_s.at[pl.ds(s * zr, zr)])
    pltpu.sync_copy(xpn_hbm.at[c, pl.ds(s * zr, zr)],
                    xs_s.at[pl.ds(s * zr, zr)])
    plsc.subcore_barrier()

    tile_edges = ep // NS
    n_chunks = tile_edges // CHUNK

    def chunk_body(k, carry):
        eoff = s * tile_edges + k * CHUNK

        @pl.when(c == 0)
        def _():
            pltpu.sync_copy(pidx_hbm.at[1, pl.ds(eoff, CHUNK)], col_v)
            pltpu.sync_copy(pidx_hbm.at[0, pl.ds(eoff, CHUNK)], row_v)
            pltpu.sync_copy(pw_hbm.at[pl.ds(eoff, CHUNK)], w_v)

        @pl.when(c == 1)
        def _():
            pltpu.sync_copy(nidx_hbm.at[1, pl.ds(eoff, CHUNK)], col_v)
            pltpu.sync_copy(nidx_hbm.at[0, pl.ds(eoff, CHUNK)], row_v)
            pltpu.sync_copy(nw_hbm.at[pl.ds(eoff, CHUNK)], w_v)

        pltpu.async_copy(xs_s.at[col_v], gath_v, gsem).wait()

        def scale_body(g, carry2):
            wgrp = w_v[pl.ds(g * LANES, LANES)]
            for e in range(LANES):
                ws = _vbroadcast(wgrp, e)
                r = g * LANES + e
                gath_v[r, 0:16] = gath_v[r, 0:16] * ws
                gath_v[r, 16:32] = gath_v[r, 16:32] * ws
            return carry2

        lax.fori_loop(0, CHUNK // LANES, scale_body, 0)

        pltpu.async_copy(gath_v, acc_s.at[row_v], ssem, add=True).wait()
        return carry

    lax.fori_loop(0, n_chunks, chunk_body, 0)
    plsc.subcore_barrier()

    pltpu.sync_copy(acc_s.at[pl.ds(s * zr, zr)],
                    out_hbm.at[c, pl.ds(s * zr, zr)])


def kernel(x, pos_index, pos_weight, neg_index, neg_weight, other_index,
           other_weight, W_org, W_pos, W_neg, W_mlp, g_org, b_org, g_pos,
           b_pos, g_neg, b_neg, prelu_a):
    n, da = x.shape
    e = pos_index.shape[1]
    f32 = jnp.float32

    wcat = jnp.concatenate([W_org, W_pos, W_neg], axis=1)
    xo, xpn = pl.pallas_call(
        _proj_body,
        out_shape=(jax.ShapeDtypeStruct((n, DB), f32),
                   jax.ShapeDtypeStruct((2, n, DB), f32)),
    )(x, wcat)

    step = NS * CHUNK
    ep = ((e + step - 1) // step) * step
    pad = ep - e
    if pad:
        pos_index = jnp.pad(pos_index, ((0, 0), (0, pad)))
        neg_index = jnp.pad(neg_index, ((0, 0), (0, pad)))
        pos_weight = jnp.pad(pos_weight, (0, pad))
        neg_weight = jnp.pad(neg_weight, (0, pad))
    zeros = jnp.zeros((n, DB), f32)

    mesh = plsc.VectorSubcoreMesh(core_axis_name="c", subcore_axis_name="s")
    spmm = pl.kernel(
        functools.partial(_spmm_sc_body, n, ep),
        out_type=jax.ShapeDtypeStruct((2, n, DB), f32),
        mesh=mesh,
        scratch_types=[
            pltpu.VMEM((CHUNK,), jnp.int32),
            pltpu.VMEM((CHUNK,), jnp.int32),
            pltpu.VMEM((CHUNK,), f32),
            pltpu.VMEM((CHUNK, DB), f32),
            pltpu.VMEM_SHARED((n, DB), f32),
            pltpu.VMEM_SHARED((n, DB), f32),
            pltpu.SemaphoreType.DMA,
            pltpu.SemaphoreType.DMA,
        ],
        compiler_params=pltpu.CompilerParams(use_tc_tiling_on_sc=False),
    )
    seg = spmm(xpn, pos_index, pos_weight, neg_index, neg_weight, zeros)

    gs = jnp.stack([g_org, g_pos, g_neg]).reshape(3, DB)
    bs = jnp.stack([b_org, b_pos, b_neg]).reshape(3, DB)
    embs = pl.pallas_call(
        _post_body,
        out_shape=jax.ShapeDtypeStruct((n, DB), f32),
        in_specs=[
            pl.BlockSpec(memory_space=pltpu.VMEM),
            pl.BlockSpec(memory_space=pltpu.VMEM),
            pl.BlockSpec(memory_space=pltpu.VMEM),
            pl.BlockSpec(memory_space=pltpu.VMEM),
            pl.BlockSpec(memory_space=pltpu.VMEM),
            pl.BlockSpec(memory_space=pltpu.VMEM),
            pl.BlockSpec(memory_space=pltpu.SMEM),
        ],
    )(xo, seg[0], seg[1], W_mlp, gs, bs, prelu_a.reshape(1))
    return embs
